# baseline (speedup 1.0000x reference)
import functools

import jax
import jax.numpy as jnp
from jax import lax
from jax.experimental import pallas as pl
from jax.experimental.pallas import tpu as pltpu
from jax.experimental.pallas import tpu_sc as plsc

B, N, K = 2, 2048, 16
IN_CH, CH, NB, NCLS = 6, 64, 3, 40

_HI = jax.lax.Precision.HIGHEST
_BIG = 3.0e38


_KT = 512


def _knn_body(tile_ref, full_ref, idx_ref):
    b = pl.program_id(0)
    pt = tile_ref[0]
    pf = full_ref[0]
    sqt = jnp.sum(pt * pt, axis=1, keepdims=True)
    sqf = jnp.sum(pf * pf, axis=1, keepdims=True)
    cross = lax.dot_general(pt, pf, (((1,), (1,)), ((), ())),
                            preferred_element_type=jnp.float32)
    d = sqt - 2.0 * cross + jnp.transpose(sqf)
    colid = lax.broadcasted_iota(jnp.int32, (_KT, N), 1)
    cols = []
    for _ in range(K):
        m = jnp.min(d, axis=1, keepdims=True)
        cand = jnp.where(d == m, colid, N)
        a = jnp.min(cand, axis=1, keepdims=True)
        cols.append(a)
        d = jnp.where(colid == a, _BIG, d)
    idx_ref[0] = jnp.concatenate(cols, axis=1) + b * N


def _knn_topk(pts):
    grid = (pts.shape[0], N // _KT)
    return pl.pallas_call(
        _knn_body,
        grid=grid,
        in_specs=[
            pl.BlockSpec((1, _KT, 3), lambda b, t: (b, t, 0)),
            pl.BlockSpec((1, N, 3), lambda b, t: (b, 0, 0)),
        ],
        out_specs=pl.BlockSpec((1, _KT, K), lambda b, t: (b, t, 0)),
        out_shape=jax.ShapeDtypeStruct((pts.shape[0], N, K), jnp.int32),
    )(pts, pts)


_GCH = 128
_GNB = 4
_GD = 2
_GPAD = 128


def _pad_cols(w):
    r, c = w.shape
    return jnp.concatenate([w, jnp.zeros((r, _GPAD - c), w.dtype)], axis=1)


def _sc_gather(table, idx, c):
    info = plsc.get_sparse_core_info()
    nw = info.num_cores * info.num_subcores
    e = idx.shape[0]
    per_w = e // nw
    nch = per_w // _GCH
    mesh = plsc.VectorSubcoreMesh(core_axis_name="c", subcore_axis_name="s")

    @functools.partial(
        pl.kernel,
        mesh=mesh,
        out_type=jax.ShapeDtypeStruct((e, c), jnp.float32),
        scratch_types=[
            pltpu.VMEM((per_w,), jnp.int32),
            pltpu.VMEM((_GNB, _GCH, c), jnp.float32),
        ] + [pltpu.SemaphoreType.DMA] * (2 * _GNB),
    )
    def gk(table_hbm, idx_hbm, out_hbm, idx_v, rows_v, *sems):
        wid = lax.axis_index("s") * info.num_cores + lax.axis_index("c")
        base = wid * per_w
        gsem, wsem = sems[:_GNB], sems[_GNB:]
        pltpu.sync_copy(idx_hbm.at[pl.ds(base, per_w)], idx_v)
        gh = [None] * _GNB
        wh = [None] * _GNB

        def writeback(t):
            pb = t % _GNB
            gh[pb].wait()
            wh[pb] = pltpu.async_copy(
                rows_v.at[pb],
                out_hbm.at[pl.ds(base + t * _GCH, _GCH)], wsem[pb])

        for j in range(nch):
            b = j % _GNB
            if wh[b] is not None:
                wh[b].wait()
            gh[b] = pltpu.async_copy(
                table_hbm.at[idx_v.at[pl.ds(j * _GCH, _GCH)]],
                rows_v.at[b], gsem[b])
            if j >= _GD:
                writeback(j - _GD)
        for t in range(max(nch - _GD, 0), nch):
            writeback(t)
        for b in range(_GNB):
            if wh[b] is not None:
                wh[b].wait()

    return gk(table, idx)


_MT = 1024


def _head_body(x_ref, xg_ref, w1T_ref, e1b_ref, w2T_ref, e2b_ref,
               n1T_ref, n1b_ref, n2T_ref, n2b_ref,
               e_ref, h_ref):
    xt = x_ref[0]
    xj = xg_ref[0][:, :, :IN_CH]
    xn = jnp.broadcast_to(xt[:, None, :], (_MT, K, IN_CH))
    cat = jnp.concatenate([xn, xn - xj, xn, xj], axis=2).reshape(_MT * K, 4 * IN_CH)
    e1 = jax.nn.relu(
        jnp.dot(cat, w1T_ref[...], preferred_element_type=jnp.float32)
        + e1b_ref[...])
    e2 = jax.nn.relu(
        jnp.dot(e1, w2T_ref[...], preferred_element_type=jnp.float32)
        + e2b_ref[...])
    out = e2.shape[-1]
    e_ref[0] = e2.reshape(_MT, K, out)
    msum = jnp.sum(e2.reshape(_MT, K, out), axis=1)
    zcat = jnp.concatenate([xt, msum], axis=1)
    z = jax.nn.relu(
        jnp.dot(zcat, n1T_ref[...], preferred_element_type=jnp.float32)
        + n1b_ref[...])
    h_ref[0] = jax.nn.relu(
        jnp.dot(z, n2T_ref[...], preferred_element_type=jnp.float32)
        + n2b_ref[...])


def _block_body(h_ref, e_ref, vg_ref, w1T_ref, e1b_ref, w2T_ref, e2b_ref,
                n1T_ref, n1b_ref, n2T_ref, n2b_ref, *out_refs, emit_edges):
    h = h_ref[0]
    c_in = h.shape[-1]
    hj = vg_ref[0][:, :, :c_in]
    hn = jnp.broadcast_to(h[:, None, :], (_MT, K, c_in))
    cat = jnp.concatenate([e_ref[0], hn, hj], axis=2).reshape(_MT * K, 3 * c_in)
    e1 = jax.nn.relu(
        jnp.dot(cat, w1T_ref[...], preferred_element_type=jnp.float32)
        + e1b_ref[...])
    e2 = jax.nn.relu(
        jnp.dot(e1, w2T_ref[...], preferred_element_type=jnp.float32)
        + e2b_ref[...])
    out = e2.shape[-1]
    msum = jnp.sum(e2.reshape(_MT, K, out), axis=1)
    zcat = jnp.concatenate([h, msum], axis=1)
    z = jax.nn.relu(
        jnp.dot(zcat, n1T_ref[...], preferred_element_type=jnp.float32)
        + n1b_ref[...])
    hnew = jax.nn.relu(
        jnp.dot(z, n2T_ref[...], preferred_element_type=jnp.float32)
        + n2b_ref[...])
    if emit_edges:
        e_out_ref, h_out_ref = out_refs
        e_out_ref[0] = e2.reshape(_MT, K, out)
    else:
        (h_out_ref,) = out_refs
    h_out_ref[0] = hnew


def _full_spec(shape):
    return pl.BlockSpec(shape, lambda b, t: tuple(0 for _ in shape))


def _run_head(x, xg, w):
    nb = x.shape[0]
    grid = (nb, N // _MT)
    out = w["w2T"].shape[1]
    return pl.pallas_call(
        _head_body,
        grid=grid,
        in_specs=[
            pl.BlockSpec((1, _MT, IN_CH), lambda b, t: (b, t, 0)),
            pl.BlockSpec((1, _MT, K, _GPAD), lambda b, t: (b, t, 0, 0)),
            _full_spec(w["w1T"].shape), _full_spec(w["e1b"].shape),
            _full_spec(w["w2T"].shape), _full_spec(w["e2b"].shape),
            _full_spec(w["n1T"].shape), _full_spec(w["n1b"].shape),
            _full_spec(w["n2T"].shape), _full_spec(w["n2b"].shape),
        ],
        out_specs=[
            pl.BlockSpec((1, _MT, K, out), lambda b, t: (b, t, 0, 0)),
            pl.BlockSpec((1, _MT, out), lambda b, t: (b, t, 0)),
        ],
        out_shape=[
            jax.ShapeDtypeStruct((nb, N, K, out), jnp.float32),
            jax.ShapeDtypeStruct((nb, N, out), jnp.float32),
        ],
    )(x, xg, w["w1T"], w["e1b"], w["w2T"], w["e2b"], w["n1T"], w["n1b"],
      w["n2T"], w["n2b"])


def _run_block(h, e, hg, w, emit_edges):
    nb = h.shape[0]
    grid = (nb, N // _MT)
    c_in = h.shape[-1]
    out = w["w2T"].shape[1]
    out_specs = [pl.BlockSpec((1, _MT, out), lambda b, t: (b, t, 0))]
    out_shape = [jax.ShapeDtypeStruct((nb, N, out), jnp.float32)]
    if emit_edges:
        out_specs = [
            pl.BlockSpec((1, _MT, K, out), lambda b, t: (b, t, 0, 0)),
            out_specs[0],
        ]
        out_shape = [
            jax.ShapeDtypeStruct((nb, N, K, out), jnp.float32),
            out_shape[0],
        ]
    return pl.pallas_call(
        functools.partial(_block_body, emit_edges=emit_edges),
        grid=grid,
        in_specs=[
            pl.BlockSpec((1, _MT, c_in), lambda b, t: (b, t, 0)),
            pl.BlockSpec((1, _MT, K, c_in), lambda b, t: (b, t, 0, 0)),
            pl.BlockSpec((1, _MT, K, _GPAD), lambda b, t: (b, t, 0, 0)),
            _full_spec(w["w1T"].shape), _full_spec(w["e1b"].shape),
            _full_spec(w["w2T"].shape), _full_spec(w["e2b"].shape),
            _full_spec(w["n1T"].shape), _full_spec(w["n1b"].shape),
            _full_spec(w["n2T"].shape), _full_spec(w["n2b"].shape),
        ],
        out_specs=out_specs,
        out_shape=out_shape,
    )(h, e, hg, w["w1T"], w["e1b"], w["w2T"], w["e2b"], w["n1T"], w["n1b"],
      w["n2T"], w["n2b"])


def _fusion_body(h1_ref, h2_ref, h3_ref, fT_ref, fb_ref,
                 p0T_ref, p0b_ref, p1T_ref, p1b_ref, p2T_ref, p2b_ref,
                 out_ref):
    cat = jnp.concatenate([h1_ref[0], h2_ref[0], h3_ref[0]], axis=1)
    f = jax.nn.relu(
        jnp.dot(cat, fT_ref[...], preferred_element_type=jnp.float32)
        + fb_ref[...])
    g = jnp.max(f, axis=0, keepdims=True)
    g = jax.nn.relu(
        jnp.dot(g, p0T_ref[...], preferred_element_type=jnp.float32) + p0b_ref[...])
    g = jax.nn.relu(
        jnp.dot(g, p1T_ref[...], preferred_element_type=jnp.float32) + p1b_ref[...])
    out_ref[0] = (
        jnp.dot(g, p2T_ref[...], preferred_element_type=jnp.float32) + p2b_ref[...])


def _run_fusion(h1, h2, h3, w):
    return pl.pallas_call(
        _fusion_body,
        grid=(h1.shape[0],),
        in_specs=[
            pl.BlockSpec((1, N, h1.shape[-1]), lambda b: (b, 0, 0)),
            pl.BlockSpec((1, N, h2.shape[-1]), lambda b: (b, 0, 0)),
            pl.BlockSpec((1, N, h3.shape[-1]), lambda b: (b, 0, 0)),
            pl.BlockSpec(w["fT"].shape, lambda b: (0, 0)),
            pl.BlockSpec(w["fb"].shape, lambda b: (0, 0)),
            pl.BlockSpec(w["p0T"].shape, lambda b: (0, 0)),
            pl.BlockSpec(w["p0b"].shape, lambda b: (0, 0)),
            pl.BlockSpec(w["p1T"].shape, lambda b: (0, 0)),
            pl.BlockSpec(w["p1b"].shape, lambda b: (0, 0)),
            pl.BlockSpec(w["p2T"].shape, lambda b: (0, 0)),
            pl.BlockSpec(w["p2b"].shape, lambda b: (0, 0)),
        ],
        out_specs=pl.BlockSpec((1, 1, NCLS), lambda b: (b, 0, 0)),
        out_shape=jax.ShapeDtypeStruct((h1.shape[0], 1, NCLS), jnp.float32),
    )(h1, h2, h3, w["fT"], w["fb"], w["p0T"], w["p0b"],
      w["p1T"], w["p1b"], w["p2T"], w["p2b"])


def _row(v):
    return v.reshape(1, -1)


def _prep_mp(p):
    return {
        "w1T": p["we1"].T, "e1b": _row(p["be1"]),
        "w2T": p["we2"].T, "e2b": _row(p["be2"]),
        "n1T": p["wn1"].T, "n1b": _row(p["bn1"]),
        "n2T": p["wn2"].T, "n2b": _row(p["bn2"]),
    }


def kernel(inputs, params):
    x = jnp.transpose(inputs[:, :, :, 0], (0, 2, 1))

    hw = _prep_mp(params["head"])
    b0 = _prep_mp(params["blocks"][0])
    b1 = _prep_mp(params["blocks"][1])
    fw = {
        "fT": params["fusion_w"].T, "fb": _row(params["fusion_b"]),
        "p0T": params["pred"][0]["w"].T, "p0b": _row(params["pred"][0]["b"]),
        "p1T": params["pred"][1]["w"].T, "p1b": _row(params["pred"][1]["b"]),
        "p2T": params["pred"][2]["w"].T, "p2b": _row(params["pred"][2]["b"]),
    }

    idx_flat = _knn_topk(x[:, :, 0:3]).reshape(B * N * K)

    xg = _sc_gather(_pad_cols(x.reshape(B * N, IN_CH)), idx_flat,
                    _GPAD).reshape(B, N, K, _GPAD)
    e_h, h1 = _run_head(x, xg, hw)

    h1g = _sc_gather(_pad_cols(h1.reshape(B * N, CH)), idx_flat,
                     _GPAD).reshape(B, N, K, _GPAD)
    e0, h2 = _run_block(h1, e_h, h1g, b0, emit_edges=True)

    h2g = _sc_gather(h2.reshape(B * N, 2 * CH), idx_flat,
                     2 * CH).reshape(B, N, K, 2 * CH)
    (h3,) = _run_block(h2, e0, h2g, b1, emit_edges=False)

    out = _run_fusion(h1, h2, h3, fw)
    return jnp.transpose(out, (0, 2, 1)), inputs


# speedup vs baseline: 1.0244x; 1.0244x over previous
import functools

import jax
import jax.numpy as jnp
from jax import lax
from jax.experimental import pallas as pl
from jax.experimental.pallas import tpu as pltpu
from jax.experimental.pallas import tpu_sc as plsc

B, N, K = 2, 2048, 16
IN_CH, CH, NB, NCLS = 6, 64, 3, 40

_HI = jax.lax.Precision.HIGHEST
_BIG = 3.0e38


_KT = 512


def _knn_body(tile_ref, full_ref, idx_ref):
    b = pl.program_id(0)
    pt = tile_ref[0]
    pf = full_ref[0]
    sqt = jnp.sum(pt * pt, axis=1, keepdims=True)
    sqf = jnp.sum(pf * pf, axis=1, keepdims=True)
    cross = lax.dot_general(pt, pf, (((1,), (1,)), ((), ())),
                            preferred_element_type=jnp.float32)
    d = sqt - 2.0 * cross + jnp.transpose(sqf)
    colid = lax.broadcasted_iota(jnp.int32, (_KT, N), 1)
    cols = []
    for _ in range(K):
        m = jnp.min(d, axis=1, keepdims=True)
        cand = jnp.where(d == m, colid, N)
        a = jnp.min(cand, axis=1, keepdims=True)
        cols.append(a)
        d = jnp.where(colid == a, _BIG, d)
    idx_ref[0] = jnp.concatenate(cols, axis=1) + b * N


def _knn_topk(pts):
    grid = (pts.shape[0], N // _KT)
    return pl.pallas_call(
        _knn_body,
        grid=grid,
        in_specs=[
            pl.BlockSpec((1, _KT, 3), lambda b, t: (b, t, 0)),
            pl.BlockSpec((1, N, 3), lambda b, t: (b, 0, 0)),
        ],
        out_specs=pl.BlockSpec((1, _KT, K), lambda b, t: (b, t, 0)),
        out_shape=jax.ShapeDtypeStruct((pts.shape[0], N, K), jnp.int32),
    )(pts, pts)


_GCH = 128
_GNB = 4
_GD = 2
_GPAD = 128


def _pad_cols(w):
    r, c = w.shape
    return jnp.concatenate([w, jnp.zeros((r, _GPAD - c), w.dtype)], axis=1)


def _sc_gather(table, idx, c):
    info = plsc.get_sparse_core_info()
    nw = info.num_cores * info.num_subcores
    e = idx.shape[0]
    per_w = e // nw
    nch = per_w // _GCH
    mesh = plsc.VectorSubcoreMesh(core_axis_name="c", subcore_axis_name="s")

    @functools.partial(
        pl.kernel,
        mesh=mesh,
        out_type=jax.ShapeDtypeStruct((e, c), jnp.float32),
        scratch_types=[
            pltpu.VMEM((per_w,), jnp.int32),
            pltpu.VMEM((_GNB, _GCH, c), jnp.float32),
        ] + [pltpu.SemaphoreType.DMA] * (2 * _GNB),
    )
    def gk(table_hbm, idx_hbm, out_hbm, idx_v, rows_v, *sems):
        wid = lax.axis_index("s") * info.num_cores + lax.axis_index("c")
        base = wid * per_w
        gsem, wsem = sems[:_GNB], sems[_GNB:]
        pltpu.sync_copy(idx_hbm.at[pl.ds(base, per_w)], idx_v)
        gh = [None] * _GNB
        wh = [None] * _GNB

        def writeback(t):
            pb = t % _GNB
            gh[pb].wait()
            wh[pb] = pltpu.async_copy(
                rows_v.at[pb],
                out_hbm.at[pl.ds(base + t * _GCH, _GCH)], wsem[pb])

        for j in range(nch):
            b = j % _GNB
            if wh[b] is not None:
                wh[b].wait()
            gh[b] = pltpu.async_copy(
                table_hbm.at[idx_v.at[pl.ds(j * _GCH, _GCH)]],
                rows_v.at[b], gsem[b])
            if j >= _GD:
                writeback(j - _GD)
        for t in range(max(nch - _GD, 0), nch):
            writeback(t)
        for b in range(_GNB):
            if wh[b] is not None:
                wh[b].wait()

    return gk(table, idx)


_MT = 512


def _head_body(x_ref, xg_ref, w1T_ref, e1b_ref, w2T_ref, e2b_ref,
               n1T_ref, n1b_ref, n2T_ref, n2b_ref,
               e_ref, h_ref):
    xt = x_ref[0]
    xj = xg_ref[0][:, :, :IN_CH]
    xn = jnp.broadcast_to(xt[:, None, :], (_MT, K, IN_CH))
    cat = jnp.concatenate([xn, xn - xj, xn, xj], axis=2).reshape(_MT * K, 4 * IN_CH)
    e1 = jax.nn.relu(
        jnp.dot(cat, w1T_ref[...], preferred_element_type=jnp.float32)
        + e1b_ref[...])
    e2 = jax.nn.relu(
        jnp.dot(e1, w2T_ref[...], preferred_element_type=jnp.float32)
        + e2b_ref[...])
    out = e2.shape[-1]
    e_ref[0] = e2.reshape(_MT, K, out)
    msum = jnp.sum(e2.reshape(_MT, K, out), axis=1)
    zcat = jnp.concatenate([xt, msum], axis=1)
    z = jax.nn.relu(
        jnp.dot(zcat, n1T_ref[...], preferred_element_type=jnp.float32)
        + n1b_ref[...])
    h_ref[0] = jax.nn.relu(
        jnp.dot(z, n2T_ref[...], preferred_element_type=jnp.float32)
        + n2b_ref[...])


def _block_body(h_ref, e_ref, vg_ref, w1T_ref, e1b_ref, w2T_ref, e2b_ref,
                n1T_ref, n1b_ref, n2T_ref, n2b_ref, *out_refs, emit_edges):
    h = h_ref[0]
    c_in = h.shape[-1]
    hj = vg_ref[0][:, :, :c_in]
    hn = jnp.broadcast_to(h[:, None, :], (_MT, K, c_in))
    cat = jnp.concatenate([e_ref[0], hn, hj], axis=2).reshape(_MT * K, 3 * c_in)
    e1 = jax.nn.relu(
        jnp.dot(cat, w1T_ref[...], preferred_element_type=jnp.float32)
        + e1b_ref[...])
    e2 = jax.nn.relu(
        jnp.dot(e1, w2T_ref[...], preferred_element_type=jnp.float32)
        + e2b_ref[...])
    out = e2.shape[-1]
    msum = jnp.sum(e2.reshape(_MT, K, out), axis=1)
    zcat = jnp.concatenate([h, msum], axis=1)
    z = jax.nn.relu(
        jnp.dot(zcat, n1T_ref[...], preferred_element_type=jnp.float32)
        + n1b_ref[...])
    hnew = jax.nn.relu(
        jnp.dot(z, n2T_ref[...], preferred_element_type=jnp.float32)
        + n2b_ref[...])
    if emit_edges:
        e_out_ref, h_out_ref = out_refs
        e_out_ref[0] = e2.reshape(_MT, K, out)
    else:
        (h_out_ref,) = out_refs
    h_out_ref[0] = hnew


def _full_spec(shape):
    return pl.BlockSpec(shape, lambda b, t: tuple(0 for _ in shape))


def _run_head(x, xg, w):
    nb = x.shape[0]
    grid = (nb, N // _MT)
    out = w["w2T"].shape[1]
    return pl.pallas_call(
        _head_body,
        grid=grid,
        in_specs=[
            pl.BlockSpec((1, _MT, IN_CH), lambda b, t: (b, t, 0)),
            pl.BlockSpec((1, _MT, K, _GPAD), lambda b, t: (b, t, 0, 0)),
            _full_spec(w["w1T"].shape), _full_spec(w["e1b"].shape),
            _full_spec(w["w2T"].shape), _full_spec(w["e2b"].shape),
            _full_spec(w["n1T"].shape), _full_spec(w["n1b"].shape),
            _full_spec(w["n2T"].shape), _full_spec(w["n2b"].shape),
        ],
        out_specs=[
            pl.BlockSpec((1, _MT, K, out), lambda b, t: (b, t, 0, 0)),
            pl.BlockSpec((1, _MT, out), lambda b, t: (b, t, 0)),
        ],
        out_shape=[
            jax.ShapeDtypeStruct((nb, N, K, out), jnp.float32),
            jax.ShapeDtypeStruct((nb, N, out), jnp.float32),
        ],
    )(x, xg, w["w1T"], w["e1b"], w["w2T"], w["e2b"], w["n1T"], w["n1b"],
      w["n2T"], w["n2b"])


def _run_block(h, e, hg, w, emit_edges):
    nb = h.shape[0]
    grid = (nb, N // _MT)
    c_in = h.shape[-1]
    out = w["w2T"].shape[1]
    out_specs = [pl.BlockSpec((1, _MT, out), lambda b, t: (b, t, 0))]
    out_shape = [jax.ShapeDtypeStruct((nb, N, out), jnp.float32)]
    if emit_edges:
        out_specs = [
            pl.BlockSpec((1, _MT, K, out), lambda b, t: (b, t, 0, 0)),
            out_specs[0],
        ]
        out_shape = [
            jax.ShapeDtypeStruct((nb, N, K, out), jnp.float32),
            out_shape[0],
        ]
    return pl.pallas_call(
        functools.partial(_block_body, emit_edges=emit_edges),
        grid=grid,
        in_specs=[
            pl.BlockSpec((1, _MT, c_in), lambda b, t: (b, t, 0)),
            pl.BlockSpec((1, _MT, K, c_in), lambda b, t: (b, t, 0, 0)),
            pl.BlockSpec((1, _MT, K, _GPAD), lambda b, t: (b, t, 0, 0)),
            _full_spec(w["w1T"].shape), _full_spec(w["e1b"].shape),
            _full_spec(w["w2T"].shape), _full_spec(w["e2b"].shape),
            _full_spec(w["n1T"].shape), _full_spec(w["n1b"].shape),
            _full_spec(w["n2T"].shape), _full_spec(w["n2b"].shape),
        ],
        out_specs=out_specs,
        out_shape=out_shape,
    )(h, e, hg, w["w1T"], w["e1b"], w["w2T"], w["e2b"], w["n1T"], w["n1b"],
      w["n2T"], w["n2b"])


def _fusion_body(h1_ref, h2_ref, h3_ref, fT_ref, fb_ref,
                 p0T_ref, p0b_ref, p1T_ref, p1b_ref, p2T_ref, p2b_ref,
                 out_ref):
    cat = jnp.concatenate([h1_ref[0], h2_ref[0], h3_ref[0]], axis=1)
    f = jax.nn.relu(
        jnp.dot(cat, fT_ref[...], preferred_element_type=jnp.float32)
        + fb_ref[...])
    g = jnp.max(f, axis=0, keepdims=True)
    g = jax.nn.relu(
        jnp.dot(g, p0T_ref[...], preferred_element_type=jnp.float32) + p0b_ref[...])
    g = jax.nn.relu(
        jnp.dot(g, p1T_ref[...], preferred_element_type=jnp.float32) + p1b_ref[...])
    out_ref[0] = (
        jnp.dot(g, p2T_ref[...], preferred_element_type=jnp.float32) + p2b_ref[...])


def _run_fusion(h1, h2, h3, w):
    return pl.pallas_call(
        _fusion_body,
        grid=(h1.shape[0],),
        in_specs=[
            pl.BlockSpec((1, N, h1.shape[-1]), lambda b: (b, 0, 0)),
            pl.BlockSpec((1, N, h2.shape[-1]), lambda b: (b, 0, 0)),
            pl.BlockSpec((1, N, h3.shape[-1]), lambda b: (b, 0, 0)),
            pl.BlockSpec(w["fT"].shape, lambda b: (0, 0)),
            pl.BlockSpec(w["fb"].shape, lambda b: (0, 0)),
            pl.BlockSpec(w["p0T"].shape, lambda b: (0, 0)),
            pl.BlockSpec(w["p0b"].shape, lambda b: (0, 0)),
            pl.BlockSpec(w["p1T"].shape, lambda b: (0, 0)),
            pl.BlockSpec(w["p1b"].shape, lambda b: (0, 0)),
            pl.BlockSpec(w["p2T"].shape, lambda b: (0, 0)),
            pl.BlockSpec(w["p2b"].shape, lambda b: (0, 0)),
        ],
        out_specs=pl.BlockSpec((1, 1, NCLS), lambda b: (b, 0, 0)),
        out_shape=jax.ShapeDtypeStruct((h1.shape[0], 1, NCLS), jnp.float32),
    )(h1, h2, h3, w["fT"], w["fb"], w["p0T"], w["p0b"],
      w["p1T"], w["p1b"], w["p2T"], w["p2b"])


def _row(v):
    return v.reshape(1, -1)


def _prep_mp(p):
    return {
        "w1T": p["we1"].T, "e1b": _row(p["be1"]),
        "w2T": p["we2"].T, "e2b": _row(p["be2"]),
        "n1T": p["wn1"].T, "n1b": _row(p["bn1"]),
        "n2T": p["wn2"].T, "n2b": _row(p["bn2"]),
    }


def kernel(inputs, params):
    x = jnp.transpose(inputs[:, :, :, 0], (0, 2, 1))

    hw = _prep_mp(params["head"])
    b0 = _prep_mp(params["blocks"][0])
    b1 = _prep_mp(params["blocks"][1])
    fw = {
        "fT": params["fusion_w"].T, "fb": _row(params["fusion_b"]),
        "p0T": params["pred"][0]["w"].T, "p0b": _row(params["pred"][0]["b"]),
        "p1T": params["pred"][1]["w"].T, "p1b": _row(params["pred"][1]["b"]),
        "p2T": params["pred"][2]["w"].T, "p2b": _row(params["pred"][2]["b"]),
    }

    idx_flat = _knn_topk(x[:, :, 0:3]).reshape(B * N * K)

    xg = _sc_gather(_pad_cols(x.reshape(B * N, IN_CH)), idx_flat,
                    _GPAD).reshape(B, N, K, _GPAD)
    e_h, h1 = _run_head(x, xg, hw)

    h1g = _sc_gather(_pad_cols(h1.reshape(B * N, CH)), idx_flat,
                     _GPAD).reshape(B, N, K, _GPAD)
    e0, h2 = _run_block(h1, e_h, h1g, b0, emit_edges=True)

    h2g = _sc_gather(h2.reshape(B * N, 2 * CH), idx_flat,
                     2 * CH).reshape(B, N, K, 2 * CH)
    (h3,) = _run_block(h2, e0, h2g, b1, emit_edges=False)

    out = _run_fusion(h1, h2, h3, fw)
    return jnp.transpose(out, (0, 2, 1)), inputs
